# baseline (speedup 1.0000x reference)
import functools

import jax
import jax.numpy as jnp
from jax import lax
from jax.experimental import pallas as pl
from jax.experimental.pallas import tpu as pltpu
from jax.experimental.pallas import tpu_sc as plsc

B = 16384
E = 10
S = 26
D = 16
BLK = 2048


@functools.lru_cache(maxsize=None)
def _make_sc_gather():
    info = plsc.get_sparse_core_info()
    nc, ns = info.num_cores, info.num_subcores
    nw = nc * ns
    bpw = B // nw
    mesh = plsc.VectorSubcoreMesh(core_axis_name="c", subcore_axis_name="s")

    @functools.partial(
        pl.kernel,
        mesh=mesh,
        out_type=jax.ShapeDtypeStruct((B, D), jnp.float32),
        scratch_types=[
            pltpu.VMEM((bpw,), jnp.int32),
            pltpu.VMEM((bpw, D), jnp.float32),
            pltpu.SemaphoreType.DMA,
        ],
        compiler_params=pltpu.CompilerParams(use_tc_tiling_on_sc=False),
    )
    def sc_gather(table_hbm, idx_hbm, out_hbm, idx_v, rows_v, sem):
        wid = lax.axis_index("s") * nc + lax.axis_index("c")
        base = wid * bpw
        pltpu.sync_copy(idx_hbm.at[pl.ds(base, bpw)], idx_v)
        pltpu.async_copy(table_hbm.at[idx_v], rows_v, sem).wait()
        pltpu.sync_copy(rows_v, out_hbm.at[pl.ds(base, bpw)])

    return sc_gather


def _tc_body(g_ref, d_ref, wp_ref, wd_ref, b_ref, o_ref):
    g = g_ref[...]
    logits = lax.dot_general(
        g, wp_ref[...], (((1,), (1,)), ((), ())),
        preferred_element_type=jnp.float32)
    logits = logits + d_ref[...] * wd_ref[...] + b_ref[...]
    m = jnp.max(logits, axis=1, keepdims=True)
    e = jnp.exp(logits - m)
    s = jnp.sum(e, axis=1, keepdims=True)
    o_ref[...] = logits - m - jnp.log(s)


def kernel(c_idx, delta, emb_table, W, b):
    emb_p = jnp.zeros((S, D), jnp.float32).at[:, :E].set(emb_table)
    g = _make_sc_gather()(emb_p, c_idx.astype(jnp.int32))

    wp = jnp.zeros((S, D), jnp.float32).at[:, :E].set(W[:, :E])
    wd = W[:, E].reshape(1, S)
    brow = b.reshape(1, S)

    return pl.pallas_call(
        _tc_body,
        grid=(B // BLK,),
        in_specs=[
            pl.BlockSpec((BLK, D), lambda i: (i, 0)),
            pl.BlockSpec((BLK, 1), lambda i: (i, 0)),
            pl.BlockSpec((S, D), lambda i: (0, 0)),
            pl.BlockSpec((1, S), lambda i: (0, 0)),
            pl.BlockSpec((1, S), lambda i: (0, 0)),
        ],
        out_specs=pl.BlockSpec((BLK, S), lambda i: (i, 0)),
        out_shape=jax.ShapeDtypeStruct((B, S), jnp.float32),
    )(g, delta.reshape(B, 1), wp, wd, brow)


# speedup vs baseline: 1.5178x; 1.5178x over previous
import functools

import jax
import jax.numpy as jnp
from jax import lax
from jax.experimental import pallas as pl
from jax.experimental.pallas import tpu as pltpu
from jax.experimental.pallas import tpu_sc as plsc

B = 16384
E = 10
S = 26
D = 16
K = 12
BLK = 4096
CH = 16


@functools.lru_cache(maxsize=None)
def _make_sc_gather():
    info = plsc.get_sparse_core_info()
    nc, ns = info.num_cores, info.num_subcores
    nw = nc * ns
    bpw = B // nw
    mesh = plsc.VectorSubcoreMesh(core_axis_name="c", subcore_axis_name="s")

    @functools.partial(
        pl.kernel,
        mesh=mesh,
        out_type=jax.ShapeDtypeStruct((B, D), jnp.float32),
        scratch_types=[
            pltpu.VMEM((S, D), jnp.float32),
            pltpu.VMEM((bpw,), jnp.int32),
            pltpu.VMEM((bpw,), jnp.float32),
            pltpu.VMEM((bpw, D), jnp.float32),
            pltpu.SemaphoreType.DMA,
        ],
        compiler_params=pltpu.CompilerParams(
            use_tc_tiling_on_sc=False, needs_layout_passes=False),
    )
    def sc_gather(comb_hbm, idx_hbm, dlt_hbm, out_hbm,
                  tbl_v, idx_v, dlt_v, rows_v, sem):
        wid = lax.axis_index("s") * nc + lax.axis_index("c")
        base = wid * bpw
        cp1 = pltpu.async_copy(comb_hbm.at[pl.ds(0, S)], tbl_v, sem)
        cp2 = pltpu.async_copy(idx_hbm.at[pl.ds(base, bpw)], idx_v, sem)
        cp3 = pltpu.async_copy(dlt_hbm.at[pl.ds(base, bpw)], dlt_v, sem)
        cp1.wait()
        cp2.wait()
        cp3.wait()

        iota = lax.iota(jnp.int32, CH)
        one = jnp.full((CH,), 1.0, jnp.float32)
        for k in range(bpw // CH):
            c16 = idx_v[pl.ds(k * CH, CH)]
            r16 = iota + (k * CH)
            for j in range(E):
                cj = jnp.full((CH,), j, jnp.int32)
                plsc.store_scatter(rows_v, [r16, cj],
                                   plsc.load_gather(tbl_v, [c16, cj]))
            d16 = dlt_v[pl.ds(k * CH, CH)]
            plsc.store_scatter(rows_v, [r16, jnp.full((CH,), E, jnp.int32)], d16)
            plsc.store_scatter(rows_v, [r16, jnp.full((CH,), E + 1, jnp.int32)], one)

        pltpu.sync_copy(rows_v, out_hbm.at[pl.ds(base, bpw)])

    return sc_gather


def _tc_body(g_ref, comb_ref, o_ref):
    g = g_ref[:, :K]
    wp = comb_ref[S:, :K]
    logits = lax.dot_general(
        g, wp, (((1,), (1,)), ((), ())),
        preferred_element_type=jnp.float32)
    m = jnp.max(logits, axis=1, keepdims=True)
    e = jnp.exp(logits - m)
    s = jnp.sum(e, axis=1, keepdims=True)
    o_ref[...] = logits - m - jnp.log(s)


def kernel(c_idx, delta, emb_table, W, b):
    comb = (jnp.zeros((2 * S, D), jnp.float32)
            .at[:S, :E].set(emb_table)
            .at[S:, :E + 1].set(W)
            .at[S:, E + 1].set(b))
    g = _make_sc_gather()(comb, c_idx.astype(jnp.int32), delta)

    return pl.pallas_call(
        _tc_body,
        grid=(B // BLK,),
        in_specs=[
            pl.BlockSpec((BLK, D), lambda i: (i, 0)),
            pl.BlockSpec((2 * S, D), lambda i: (0, 0)),
        ],
        out_specs=pl.BlockSpec((BLK, S), lambda i: (i, 0)),
        out_shape=jax.ShapeDtypeStruct((B, S), jnp.float32),
    )(g, comb)


# speedup vs baseline: 2.2691x; 1.4950x over previous
import functools

import jax
import jax.numpy as jnp
from jax import lax
from jax.experimental import pallas as pl
from jax.experimental.pallas import tpu as pltpu
from jax.experimental.pallas import tpu_sc as plsc

B = 16384
E = 10
S = 26
D = 16
TR = 32
BLK = 4096
CH = 16


@functools.lru_cache(maxsize=None)
def _make_sc_gather():
    info = plsc.get_sparse_core_info()
    nc, ns = info.num_cores, info.num_subcores
    nw = nc * ns
    bpw = B // nw
    mesh = plsc.VectorSubcoreMesh(core_axis_name="c", subcore_axis_name="s")

    @functools.partial(
        pl.kernel,
        mesh=mesh,
        out_type=jax.ShapeDtypeStruct((D, B), jnp.float32),
        scratch_types=[
            pltpu.VMEM((TR, D), jnp.float32),
            pltpu.VMEM((bpw,), jnp.int32),
            pltpu.VMEM((bpw,), jnp.float32),
            pltpu.VMEM((D, bpw), jnp.float32),
            pltpu.SemaphoreType.DMA,
        ],
        compiler_params=pltpu.CompilerParams(
            use_tc_tiling_on_sc=False, needs_layout_passes=False),
    )
    def sc_gather(tbl_hbm, idx_hbm, dlt_hbm, out_hbm,
                  tbl_v, idx_v, dlt_v, gt_v, sem):
        wid = lax.axis_index("s") * nc + lax.axis_index("c")
        base = wid * bpw
        cp1 = pltpu.async_copy(tbl_hbm, tbl_v, sem)
        cp2 = pltpu.async_copy(idx_hbm.at[pl.ds(base, bpw)], idx_v, sem)
        cp3 = pltpu.async_copy(dlt_hbm.at[pl.ds(base, bpw)], dlt_v, sem)
        cp1.wait()
        cp2.wait()
        cp3.wait()

        for k in range(bpw // CH):
            c16 = idx_v[pl.ds(k * CH, CH)]
            for j in range(E):
                cj = jnp.full((CH,), j, jnp.int32)
                gt_v[j, pl.ds(k * CH, CH)] = plsc.load_gather(tbl_v, [c16, cj])
            gt_v[E, pl.ds(k * CH, CH)] = dlt_v[pl.ds(k * CH, CH)]

        pltpu.sync_copy(gt_v, out_hbm.at[:, pl.ds(base, bpw)])

    return sc_gather


def _tc_body(g_ref, w_ref, b_ref, o_ref):
    logits = lax.dot_general(
        w_ref[...], g_ref[:E + 1, :], (((1,), (0,)), ((), ())),
        preferred_element_type=jnp.float32)
    logits = logits + b_ref[...]
    m = jnp.max(logits, axis=0, keepdims=True)
    e = jnp.exp(logits - m)
    s = jnp.sum(e, axis=0, keepdims=True)
    o_ref[...] = logits - m - jnp.log(s)


def kernel(c_idx, delta, emb_table, W, b):
    tbl = jnp.zeros((TR, D), jnp.float32).at[:S, :E].set(emb_table)
    g_t = _make_sc_gather()(tbl, c_idx.astype(jnp.int32), delta)

    out_t = pl.pallas_call(
        _tc_body,
        grid=(B // BLK,),
        in_specs=[
            pl.BlockSpec((D, BLK), lambda i: (0, i)),
            pl.BlockSpec((S, E + 1), lambda i: (0, 0)),
            pl.BlockSpec((S, 1), lambda i: (0, 0)),
        ],
        out_specs=pl.BlockSpec((S, BLK), lambda i: (0, i)),
        out_shape=jax.ShapeDtypeStruct((S, B), jnp.float32),
    )(g_t, W, b.reshape(S, 1))
    return out_t.T


# speedup vs baseline: 2.4502x; 1.0798x over previous
import functools

import jax
import jax.numpy as jnp
from jax import lax
from jax.experimental import pallas as pl
from jax.experimental.pallas import tpu as pltpu
from jax.experimental.pallas import tpu_sc as plsc

B = 16384
E = 10
S = 26
D = 16
TR = 32
BLK = 4096
CH = 16
NG = B // 128


@functools.lru_cache(maxsize=None)
def _make_sc_gather():
    info = plsc.get_sparse_core_info()
    nc, ns = info.num_cores, info.num_subcores
    nw = nc * ns
    bpw = B // nw
    gpw = bpw // 128
    mesh = plsc.VectorSubcoreMesh(core_axis_name="c", subcore_axis_name="s")

    @functools.partial(
        pl.kernel,
        mesh=mesh,
        out_type=jax.ShapeDtypeStruct((NG * D, 128), jnp.float32),
        scratch_types=[
            pltpu.VMEM((TR, D), jnp.float32),
            pltpu.VMEM((bpw,), jnp.int32),
            pltpu.VMEM((bpw,), jnp.float32),
            pltpu.VMEM((gpw * D, 128), jnp.float32),
            pltpu.SemaphoreType.DMA,
        ],
        compiler_params=pltpu.CompilerParams(
            use_tc_tiling_on_sc=False, needs_layout_passes=False),
    )
    def sc_gather(tbl_hbm, idx_hbm, dlt_hbm, out_hbm,
                  tbl_v, idx_v, dlt_v, gt_v, sem):
        wid = lax.axis_index("s") * nc + lax.axis_index("c")
        base = wid * bpw
        cp1 = pltpu.async_copy(tbl_hbm, tbl_v, sem)
        cp2 = pltpu.async_copy(idx_hbm.at[pl.ds(base, bpw)], idx_v, sem)
        cp3 = pltpu.async_copy(dlt_hbm.at[pl.ds(base, bpw)], dlt_v, sem)
        cp1.wait()
        cp2.wait()
        cp3.wait()

        for k in range(bpw // CH):
            c16 = idx_v[pl.ds(k * CH, CH)]
            row0 = D * (k // 8)
            lane0 = (k % 8) * CH
            for j in range(E):
                cj = jnp.full((CH,), j, jnp.int32)
                gt_v[row0 + j, pl.ds(lane0, CH)] = (
                    plsc.load_gather(tbl_v, [c16, cj]))
            gt_v[row0 + E, pl.ds(lane0, CH)] = dlt_v[pl.ds(k * CH, CH)]

        pltpu.sync_copy(gt_v, out_hbm.at[pl.ds(wid * gpw * D, gpw * D)])

    return sc_gather


def _tc_body(g_ref, w_ref, b_ref, o_ref):
    gt = jnp.concatenate(
        [g_ref[D * c:D * c + E + 1, :] for c in range(BLK // 128)],
        axis=1)
    logits = lax.dot_general(
        w_ref[...], gt, (((1,), (0,)), ((), ())),
        preferred_element_type=jnp.float32)
    logits = logits + b_ref[...]
    m = jnp.max(logits, axis=0, keepdims=True)
    e = jnp.exp(logits - m)
    s = jnp.sum(e, axis=0, keepdims=True)
    o_ref[...] = logits - m - jnp.log(s)


def kernel(c_idx, delta, emb_table, W, b):
    tbl = jnp.zeros((TR, D), jnp.float32).at[:S, :E].set(emb_table)
    g_s = _make_sc_gather()(tbl, c_idx.astype(jnp.int32), delta)

    out_t = pl.pallas_call(
        _tc_body,
        grid=(B // BLK,),
        in_specs=[
            pl.BlockSpec((BLK // 128 * D, 128), lambda i: (i, 0)),
            pl.BlockSpec((S, E + 1), lambda i: (0, 0)),
            pl.BlockSpec((S, 1), lambda i: (0, 0)),
        ],
        out_specs=pl.BlockSpec((S, BLK), lambda i: (0, i)),
        out_shape=jax.ShapeDtypeStruct((S, B), jnp.float32),
    )(g_s, W, b.reshape(S, 1))
    return out_t.T


# speedup vs baseline: 2.4536x; 1.0014x over previous
import functools

import jax
import jax.numpy as jnp
from jax import lax
from jax.experimental import pallas as pl
from jax.experimental.pallas import tpu as pltpu
from jax.experimental.pallas import tpu_sc as plsc

B = 16384
E = 10
S = 26
D = 16
TR = 32
BLK = 4096
CH = 16
NG = B // 128


@functools.lru_cache(maxsize=None)
def _make_sc_gather():
    info = plsc.get_sparse_core_info()
    nc, ns = info.num_cores, info.num_subcores
    nw = nc * ns
    bpw = B // nw
    gpw = bpw // 128
    mesh = plsc.VectorSubcoreMesh(core_axis_name="c", subcore_axis_name="s")

    @functools.partial(
        pl.kernel,
        mesh=mesh,
        out_type=jax.ShapeDtypeStruct((NG * D, 128), jnp.float32),
        scratch_types=[
            pltpu.VMEM((TR, D), jnp.float32),
            pltpu.VMEM((bpw,), jnp.int32),
            pltpu.VMEM((bpw,), jnp.float32),
            pltpu.VMEM((gpw * D, 128), jnp.float32),
            pltpu.SemaphoreType.DMA,
        ],
        compiler_params=pltpu.CompilerParams(
            use_tc_tiling_on_sc=False, needs_layout_passes=False,
            skip_device_barrier=True),
    )
    def sc_gather(tbl_hbm, idx_hbm, dlt_hbm, out_hbm,
                  tbl_v, idx_v, dlt_v, gt_v, sem):
        wid = lax.axis_index("s") * nc + lax.axis_index("c")
        base = wid * bpw
        cp1 = pltpu.async_copy(tbl_hbm, tbl_v, sem)
        cp2 = pltpu.async_copy(idx_hbm.at[pl.ds(base, bpw)], idx_v, sem)
        cp3 = pltpu.async_copy(dlt_hbm.at[pl.ds(base, bpw)], dlt_v, sem)
        cp1.wait()
        cp2.wait()
        cp3.wait()

        for k in range(bpw // CH):
            c16 = idx_v[pl.ds(k * CH, CH)]
            row0 = D * (k // 8)
            lane0 = (k % 8) * CH
            for j in range(E):
                cj = jnp.full((CH,), j, jnp.int32)
                gt_v[row0 + j, pl.ds(lane0, CH)] = (
                    plsc.load_gather(tbl_v, [c16, cj]))
            gt_v[row0 + E, pl.ds(lane0, CH)] = dlt_v[pl.ds(k * CH, CH)]

        pltpu.sync_copy(gt_v, out_hbm.at[pl.ds(wid * gpw * D, gpw * D)])

    return sc_gather


def _tc_body(g_ref, w_ref, b_ref, o_ref):
    gt = jnp.concatenate(
        [g_ref[D * c:D * c + E + 1, :] for c in range(BLK // 128)],
        axis=1)
    logits = lax.dot_general(
        w_ref[...], gt, (((1,), (0,)), ((), ())),
        preferred_element_type=jnp.float32)
    logits = logits + b_ref[...]
    m = jnp.max(logits, axis=0, keepdims=True)
    e = jnp.exp(logits - m)
    s = jnp.sum(e, axis=0, keepdims=True)
    o_ref[...] = logits - m - jnp.log(s)


def kernel(c_idx, delta, emb_table, W, b):
    tbl = jnp.zeros((TR, D), jnp.float32).at[:S, :E].set(emb_table)
    g_s = _make_sc_gather()(tbl, c_idx.astype(jnp.int32), delta)

    out_t = pl.pallas_call(
        _tc_body,
        grid=(B // BLK,),
        in_specs=[
            pl.BlockSpec((BLK // 128 * D, 128), lambda i: (i, 0)),
            pl.BlockSpec((S, E + 1), lambda i: (0, 0)),
            pl.BlockSpec((S, 1), lambda i: (0, 0)),
        ],
        out_specs=pl.BlockSpec((S, BLK), lambda i: (0, i)),
        out_shape=jax.ShapeDtypeStruct((S, B), jnp.float32),
    )(g_s, W, b.reshape(S, 1))
    return out_t.T


# speedup vs baseline: 2.5922x; 1.0565x over previous
import functools

import jax
import jax.numpy as jnp
from jax import lax
from jax.experimental import pallas as pl
from jax.experimental.pallas import tpu as pltpu
from jax.experimental.pallas import tpu_sc as plsc

B = 16384
E = 10
S = 26
D = 16
TR = 32
BLK = 4096
CH = 16
NG = B // 128


@functools.lru_cache(maxsize=None)
def _make_sc_gather():
    info = plsc.get_sparse_core_info()
    nc, ns = info.num_cores, info.num_subcores
    nw = nc * ns
    bpw = B // nw
    gpw = bpw // 128
    mesh = plsc.VectorSubcoreMesh(core_axis_name="c", subcore_axis_name="s")

    @functools.partial(
        pl.kernel,
        mesh=mesh,
        out_type=jax.ShapeDtypeStruct((NG * D, 128), jnp.float32),
        scratch_types=[
            pltpu.VMEM((TR, D), jnp.float32),
            pltpu.VMEM((bpw,), jnp.int32),
            pltpu.VMEM((bpw,), jnp.float32),
            pltpu.VMEM((gpw * D, 128), jnp.float32),
            pltpu.SemaphoreType.DMA,
        ],
        compiler_params=pltpu.CompilerParams(
            use_tc_tiling_on_sc=False, needs_layout_passes=False),
    )
    def sc_gather(tbl_hbm, idx_hbm, dlt_hbm, out_hbm,
                  tbl_v, idx_v, dlt_v, gt_v, sem):
        wid = lax.axis_index("s") * nc + lax.axis_index("c")
        base = wid * bpw
        cp1 = pltpu.async_copy(tbl_hbm, tbl_v, sem)
        cp2 = pltpu.async_copy(idx_hbm.at[pl.ds(base, bpw)], idx_v, sem)
        cp3 = pltpu.async_copy(dlt_hbm.at[pl.ds(base, bpw)], dlt_v, sem)
        cp1.wait()
        cp2.wait()
        cp3.wait()

        def chunk(k, carry):
            c16 = idx_v[pl.ds(k * CH, CH)]
            row0 = D * (k // 8)
            lane0 = (k % 8) * CH
            for j in range(E):
                cj = jnp.full((CH,), j, jnp.int32)
                gt_v[row0 + j, pl.ds(lane0, CH)] = (
                    plsc.load_gather(tbl_v, [c16, cj]))
            gt_v[row0 + E, pl.ds(lane0, CH)] = dlt_v[pl.ds(k * CH, CH)]
            return carry

        lax.fori_loop(0, bpw // CH, chunk, 0, unroll=4)

        pltpu.sync_copy(gt_v, out_hbm.at[pl.ds(wid * gpw * D, gpw * D)])

    return sc_gather


def _tc_body(g_ref, w_ref, b_ref, o_ref):
    gt = jnp.concatenate(
        [g_ref[D * c:D * c + E + 1, :] for c in range(BLK // 128)],
        axis=1)
    logits = lax.dot_general(
        w_ref[...], gt, (((1,), (0,)), ((), ())),
        preferred_element_type=jnp.float32)
    logits = logits + b_ref[...]
    m = jnp.max(logits, axis=0, keepdims=True)
    e = jnp.exp(logits - m)
    s = jnp.sum(e, axis=0, keepdims=True)
    o_ref[...] = logits - m - jnp.log(s)


def kernel(c_idx, delta, emb_table, W, b):
    tbl = jnp.zeros((TR, D), jnp.float32).at[:S, :E].set(emb_table)
    g_s = _make_sc_gather()(tbl, c_idx.astype(jnp.int32), delta)

    out_t = pl.pallas_call(
        _tc_body,
        grid=(B // BLK,),
        in_specs=[
            pl.BlockSpec((BLK // 128 * D, 128), lambda i: (i, 0)),
            pl.BlockSpec((S, E + 1), lambda i: (0, 0)),
            pl.BlockSpec((S, 1), lambda i: (0, 0)),
        ],
        out_specs=pl.BlockSpec((S, BLK), lambda i: (0, i)),
        out_shape=jax.ShapeDtypeStruct((S, B), jnp.float32),
    )(g_s, W, b.reshape(S, 1))
    return out_t.T
